# baseline (speedup 1.0000x reference)
import jax
import jax.numpy as jnp
from jax import lax
from jax.experimental import pallas as pl
from jax.experimental.pallas import tpu as pltpu
from jax.experimental.pallas import tpu_sc as plsc

D = 768
H = 12
DH = 64
E = 64
F = 3072
TB = 128
NG = 104
NTOK = 4096

SC_NC = 2
SC_NS = 16
SC_NW = SC_NC * SC_NS
SC_ROWS = NTOK // SC_NW

_F32 = jnp.float32


def _qkv_kernel(x_ref, wq_ref, wk_ref, wv_ref, g_ref, b_ref,
                q_ref, k_ref, v_ref, wqb, wkb, wvb):
    @pl.when(pl.program_id(0) == 0)
    def _cast():
        wqb[...] = wq_ref[...].astype(jnp.bfloat16)
        wkb[...] = wk_ref[...].astype(jnp.bfloat16)
        wvb[...] = wv_ref[...].astype(jnp.bfloat16)

    x = x_ref[...]
    mu = jnp.mean(x, axis=1, keepdims=True)
    d = x - mu
    var = jnp.mean(d * d, axis=1, keepdims=True)
    h = d * lax.rsqrt(var + 1e-5) * g_ref[...] + b_ref[...]
    hb = h.astype(jnp.bfloat16)
    q_ref[...] = lax.dot_general(wqb[...], hb, (((0,), (1,)), ((), ())),
                                 preferred_element_type=_F32)
    k_ref[...] = lax.dot_general(wkb[...], hb, (((0,), (1,)), ((), ())),
                                 preferred_element_type=_F32)
    v_ref[...] = lax.dot_general(wvb[...], hb, (((0,), (1,)), ((), ())),
                                 preferred_element_type=_F32)


def _qkv(x2d, Wq, Wk, Wv, g, b):
    n_blk = NTOK // 256
    spec_rows = pl.BlockSpec((256, D), lambda i: (i, 0))
    spec_cols = pl.BlockSpec((D, 256), lambda i: (0, i))
    spec_w = pl.BlockSpec((D, D), lambda i: (0, 0))
    spec_vec = pl.BlockSpec((1, D), lambda i: (0, 0))
    return pl.pallas_call(
        _qkv_kernel,
        grid=(n_blk,),
        in_specs=[spec_rows, spec_w, spec_w, spec_w, spec_vec, spec_vec],
        out_specs=[spec_cols, spec_cols, spec_cols],
        out_shape=[jax.ShapeDtypeStruct((D, NTOK), _F32)] * 3,
        scratch_shapes=[pltpu.VMEM((D, D), jnp.bfloat16)] * 3,
    )(x2d, Wq, Wk, Wv, g.reshape(1, D), b.reshape(1, D))


def _attn_kernel(q_ref, k_ref, v_ref, o_ref):
    s = lax.dot_general((q_ref[...] * (1.0 / 8.0)).astype(jnp.bfloat16),
                        k_ref[...].astype(jnp.bfloat16),
                        (((0,), (0,)), ((), ())),
                        preferred_element_type=_F32).astype(jnp.bfloat16)
    m = jnp.max(s, axis=1, keepdims=True)
    p = jnp.exp(s - m)
    l = jnp.sum(p.astype(_F32), axis=1, keepdims=True)
    o = lax.dot_general(v_ref[...].astype(jnp.bfloat16), p,
                        (((1,), (1,)), ((), ())),
                        preferred_element_type=_F32)
    o_ref[...] = o * jnp.transpose(1.0 / l)


def _attention(q, k, v):
    QB = 2048
    nqb = 2048 // QB
    return pl.pallas_call(
        _attn_kernel,
        grid=(2, H, nqb),
        in_specs=[
            pl.BlockSpec((DH, QB), lambda bb, hh, qq: (hh, bb * nqb + qq)),
            pl.BlockSpec((DH, 2048), lambda bb, hh, qq: (hh, bb)),
            pl.BlockSpec((DH, 2048), lambda bb, hh, qq: (hh, bb)),
        ],
        out_specs=pl.BlockSpec((DH, QB), lambda bb, hh, qq: (hh, bb * nqb + qq)),
        out_shape=jax.ShapeDtypeStruct((D, NTOK), _F32),
    )(q, k, v)


def _post_kernel(a_ref, x_ref, wo_ref, rw_ref, g_ref, b_ref,
                 x2_ref, mi_ref, lg_ref, wob):
    @pl.when(pl.program_id(0) == 0)
    def _cast():
        wob[...] = wo_ref[...].astype(jnp.bfloat16)

    i = pl.program_id(0)
    ablk = a_ref[:, pl.ds(i * 256, 256)]
    x2 = x_ref[...] + lax.dot_general(ablk.astype(jnp.bfloat16),
                                      wob[...],
                                      (((0,), (0,)), ((), ())),
                                      preferred_element_type=_F32)
    x2_ref[...] = x2
    mu = jnp.mean(x2, axis=1, keepdims=True)
    d = x2 - mu
    var = jnp.mean(d * d, axis=1, keepdims=True)
    mi = d * lax.rsqrt(var + 1e-5) * g_ref[...] + b_ref[...]
    mi_ref[...] = mi
    lg_ref[...] = jnp.dot(mi, rw_ref[...], preferred_element_type=_F32)


def _post_attn(attn, x2d, Wo, router_w, g, b):
    n_blk = NTOK // 256
    spec_rows = pl.BlockSpec((256, D), lambda i: (i, 0))
    return pl.pallas_call(
        _post_kernel,
        grid=(n_blk,),
        in_specs=[
            pl.BlockSpec((D, NTOK), lambda i: (0, 0)),
            spec_rows,
            pl.BlockSpec((D, D), lambda i: (0, 0)),
            pl.BlockSpec((D, E), lambda i: (0, 0)),
            pl.BlockSpec((1, D), lambda i: (0, 0)),
            pl.BlockSpec((1, D), lambda i: (0, 0)),
        ],
        out_specs=[spec_rows, spec_rows, pl.BlockSpec((256, E), lambda i: (i, 0))],
        out_shape=[
            jax.ShapeDtypeStruct((NTOK, D), _F32),
            jax.ShapeDtypeStruct((NTOK, D), _F32),
            jax.ShapeDtypeStruct((NTOK, E), _F32),
        ],
        scratch_shapes=[pltpu.VMEM((D, D), jnp.bfloat16)],
    )(attn, x2d, Wo, router_w, g.reshape(1, D), b.reshape(1, D))


def _route_kernel(lg_ref, pos_ref, e_ref, rs_ref, lo_ref, hi_ref):
    nblk = NTOK // TB
    lane = lax.broadcasted_iota(jnp.int32, (TB, E), 1).astype(_F32)
    ii = lax.broadcasted_iota(jnp.int32, (TB, TB), 0)
    jj = lax.broadcasted_iota(jnp.int32, (TB, TB), 1)
    tril = (jj < ii).astype(_F32)
    carry = jnp.zeros((1, E), _F32)
    ranks = []
    onehots = []
    for bi in range(nblk):
        lg = lg_ref[pl.ds(bi * TB, TB), :]
        m = jnp.max(lg, axis=1, keepdims=True)
        cand = jnp.where(lg >= m, lane, 1e9)
        eid = jnp.min(cand, axis=1, keepdims=True)
        oh = (lane == eid).astype(_F32)
        excl = jnp.dot(tril, oh, preferred_element_type=_F32) + carry
        ranks.append(jnp.sum(excl * oh, axis=1, keepdims=True))
        onehots.append(oh)
        carry = carry + jnp.sum(oh, axis=0, keepdims=True)
    counts = carry
    ei = lax.broadcasted_iota(jnp.int32, (E, E), 0)
    ej = lax.broadcasted_iota(jnp.int32, (E, E), 1)
    upper = (ei < ej).astype(_F32)
    offs = jnp.dot(counts, upper, preferred_element_type=_F32)
    for bi in range(nblk):
        off_tok = jnp.sum(onehots[bi] * offs, axis=1, keepdims=True)
        pos_ref[pl.ds(bi * TB, TB), :] = (ranks[bi] + off_tok).astype(jnp.int32)
    astart = jnp.floor(offs * 0.125) * 8.0
    span = (offs - astart) + counts
    nb = jnp.floor((span + (TB - 1)) * (1.0 / TB))
    boff = jnp.dot(nb, upper, preferred_element_type=_F32)
    g_col = lax.broadcasted_iota(jnp.int32, (NG, 1), 0).astype(_F32)
    ind = jnp.logical_and(g_col >= boff, g_col < boff + nb).astype(_F32)
    erow = lax.broadcasted_iota(jnp.int32, (1, E), 1).astype(_F32)
    cnt_g = jnp.sum(ind * counts, axis=1, keepdims=True)
    offs_g = jnp.sum(ind * offs, axis=1, keepdims=True)
    astart_g = jnp.sum(ind * astart, axis=1, keepdims=True)
    boff_g = jnp.sum(ind * boff, axis=1, keepdims=True)
    e_g = jnp.sum(ind * erow, axis=1, keepdims=True)
    k_g = g_col - boff_g
    win = astart_g + TB * k_g
    rstart = jnp.minimum(win, float(NTOK - TB))
    lo = jnp.maximum(offs_g, win) - rstart
    hi = jnp.minimum(offs_g + cnt_g, win + TB) - rstart
    hi = jnp.maximum(hi, lo)
    e_ref[...] = e_g.astype(jnp.int32)
    rs_ref[...] = rstart.astype(jnp.int32)
    lo_ref[...] = lo.astype(jnp.int32)
    hi_ref[...] = hi.astype(jnp.int32)


def _route(logits):
    return pl.pallas_call(
        _route_kernel,
        out_shape=[
            jax.ShapeDtypeStruct((NTOK, 1), jnp.int32),
            jax.ShapeDtypeStruct((NG, 1), jnp.int32),
            jax.ShapeDtypeStruct((NG, 1), jnp.int32),
            jax.ShapeDtypeStruct((NG, 1), jnp.int32),
            jax.ShapeDtypeStruct((NG, 1), jnp.int32),
        ],
    )(logits)


def _sc_scatter_rows(src, pos):
    mesh = plsc.VectorSubcoreMesh(core_axis_name="c", subcore_axis_name="s")

    def body(src_hbm, pos_hbm, out_hbm, idx_v, rows_v, sem):
        wid = lax.axis_index("s") * SC_NC + lax.axis_index("c")
        base = wid * SC_ROWS
        pltpu.sync_copy(pos_hbm.at[pl.ds(base, SC_ROWS)], idx_v)
        pltpu.sync_copy(src_hbm.at[pl.ds(base, SC_ROWS)], rows_v)
        pltpu.async_copy(rows_v, out_hbm.at[idx_v], sem).wait()

    return pl.kernel(
        body,
        out_type=jax.ShapeDtypeStruct((NTOK, D), _F32),
        mesh=mesh,
        scratch_types=[
            pltpu.VMEM((SC_ROWS,), jnp.int32),
            pltpu.VMEM((SC_ROWS, D), _F32),
            pltpu.SemaphoreType.DMA,
        ],
    )(src, pos)


def _sc_gather_rows(table, pos):
    mesh = plsc.VectorSubcoreMesh(core_axis_name="c", subcore_axis_name="s")

    def body(tab_hbm, pos_hbm, out_hbm, idx_v, rows_v, sem):
        wid = lax.axis_index("s") * SC_NC + lax.axis_index("c")
        base = wid * SC_ROWS
        pltpu.sync_copy(pos_hbm.at[pl.ds(base, SC_ROWS)], idx_v)
        pltpu.async_copy(tab_hbm.at[idx_v], rows_v, sem).wait()
        pltpu.sync_copy(rows_v, out_hbm.at[pl.ds(base, SC_ROWS)])

    return pl.kernel(
        body,
        out_type=jax.ShapeDtypeStruct((NTOK, D), _F32),
        mesh=mesh,
        scratch_types=[
            pltpu.VMEM((SC_ROWS,), jnp.int32),
            pltpu.VMEM((SC_ROWS, D), _F32),
            pltpu.SemaphoreType.DMA,
        ],
    )(table, pos)


def _ffn_kernel(e_ref, rs_ref, lo_ref, hi_ref,
                ts_ref, w1_ref, b1_ref, w2_ref, b2_ref, out_ref):
    g = pl.program_id(0)

    @pl.when(g == 0)
    def _init():
        out_ref[...] = jnp.zeros((NTOK, D), _F32)

    @pl.when(hi_ref[g] > lo_ref[g])
    def _body():
        rs = pl.multiple_of(rs_ref[g], 8)
        xblk = ts_ref[pl.ds(rs, TB), :].astype(jnp.bfloat16)
        h = jnp.dot(xblk, w1_ref[0].astype(jnp.bfloat16),
                    preferred_element_type=_F32) + b1_ref[0]
        h = jax.nn.gelu(h).astype(jnp.bfloat16)
        y = jnp.dot(h, w2_ref[0].astype(jnp.bfloat16),
                    preferred_element_type=_F32) + b2_ref[0]
        rows = lax.broadcasted_iota(jnp.int32, (TB, 1), 0)
        mask = jnp.logical_and(rows >= lo_ref[g], rows < hi_ref[g])
        y = jnp.where(mask, y, 0.0)
        out_ref[pl.ds(rs, TB), :] = out_ref[pl.ds(rs, TB), :] + y


def _grouped_ffn(ts, We1, be1, We2, be2, e_arr, rs_arr, lo_arr, hi_arr):
    grid_spec = pltpu.PrefetchScalarGridSpec(
        num_scalar_prefetch=4,
        grid=(NG,),
        in_specs=[
            pl.BlockSpec((NTOK, D), lambda g, e, r, lo, hi: (0, 0)),
            pl.BlockSpec((1, D, F), lambda g, e, r, lo, hi: (e[g], 0, 0)),
            pl.BlockSpec((1, 1, F), lambda g, e, r, lo, hi: (e[g], 0, 0)),
            pl.BlockSpec((1, F, D), lambda g, e, r, lo, hi: (e[g], 0, 0)),
            pl.BlockSpec((1, 1, D), lambda g, e, r, lo, hi: (e[g], 0, 0)),
        ],
        out_specs=pl.BlockSpec((NTOK, D), lambda g, e, r, lo, hi: (0, 0)),
    )
    return pl.pallas_call(
        _ffn_kernel,
        grid_spec=grid_spec,
        out_shape=jax.ShapeDtypeStruct((NTOK, D), _F32),
        compiler_params=pltpu.CompilerParams(vmem_limit_bytes=100 * 2**20),
    )(e_arr, rs_arr, lo_arr, hi_arr, ts,
      We1, be1.reshape(E, 1, F), We2, be2.reshape(E, 1, D))


def _final_kernel(x2_ref, gth_ref, lg_ref, out_ref):
    lg = lg_ref[...]
    m = jnp.max(lg, axis=1, keepdims=True)
    w = 1.0 / jnp.sum(jnp.exp(lg - m), axis=1, keepdims=True)
    out_ref[...] = x2_ref[...] + w * gth_ref[...]


def _finalize(x2, gathered, logits):
    n_blk = NTOK // 256
    spec_rows = pl.BlockSpec((256, D), lambda i: (i, 0))
    return pl.pallas_call(
        _final_kernel,
        grid=(n_blk,),
        in_specs=[spec_rows, spec_rows, pl.BlockSpec((256, E), lambda i: (i, 0))],
        out_specs=spec_rows,
        out_shape=jax.ShapeDtypeStruct((NTOK, D), _F32),
    )(x2, gathered, logits)


def kernel(x, Wq, Wk, Wv, Wo, ln1_g, ln1_b, ln2_g, ln2_b,
           router_w, We1, be1, We2, be2):
    x2d = x.reshape(NTOK, D)
    q, k, v = _qkv(x2d, Wq, Wk, Wv, ln1_g, ln1_b)
    attn = _attention(q, k, v)
    x2, moe_in, logits = _post_attn(attn, x2d, Wo, router_w, ln2_g, ln2_b)
    pos, e_t, rs_t, lo_t, hi_t = _route(logits)
    pos1d = pos.reshape(NTOK)
    ts = _sc_scatter_rows(moe_in, pos1d)
    ffn = _grouped_ffn(ts, We1, be1, We2, be2,
                       e_t.reshape(NG), rs_t.reshape(NG),
                       lo_t.reshape(NG), hi_t.reshape(NG))
    gathered = _sc_gather_rows(ffn, pos1d)
    out = _finalize(x2, gathered, logits)
    return out.reshape(2, 2048, D)


# speedup vs baseline: 1.0035x; 1.0035x over previous
import jax
import jax.numpy as jnp
from jax import lax
from jax.experimental import pallas as pl
from jax.experimental.pallas import tpu as pltpu
from jax.experimental.pallas import tpu_sc as plsc

D = 768
H = 12
DH = 64
E = 64
F = 3072
TB = 128
NG = 104
NTOK = 4096

SC_NC = 2
SC_NS = 16
SC_NW = SC_NC * SC_NS
SC_ROWS = NTOK // SC_NW

_F32 = jnp.float32


def _qkv_kernel(x_ref, wq_ref, wk_ref, wv_ref, g_ref, b_ref,
                q_ref, k_ref, v_ref, wqb, wkb, wvb):
    @pl.when(pl.program_id(0) == 0)
    def _cast():
        wqb[...] = wq_ref[...].astype(jnp.bfloat16)
        wkb[...] = wk_ref[...].astype(jnp.bfloat16)
        wvb[...] = wv_ref[...].astype(jnp.bfloat16)

    x = x_ref[...]
    mu = jnp.mean(x, axis=1, keepdims=True)
    d = x - mu
    var = jnp.mean(d * d, axis=1, keepdims=True)
    h = d * lax.rsqrt(var + 1e-5) * g_ref[...] + b_ref[...]
    hb = h.astype(jnp.bfloat16)
    q_ref[...] = lax.dot_general(wqb[...], hb, (((0,), (1,)), ((), ())),
                                 preferred_element_type=_F32)
    k_ref[...] = lax.dot_general(wkb[...], hb, (((0,), (1,)), ((), ())),
                                 preferred_element_type=_F32)
    v_ref[...] = lax.dot_general(wvb[...], hb, (((0,), (1,)), ((), ())),
                                 preferred_element_type=_F32)


def _qkv(x2d, Wq, Wk, Wv, g, b):
    n_blk = NTOK // 256
    spec_rows = pl.BlockSpec((256, D), lambda i: (i, 0))
    spec_cols = pl.BlockSpec((D, 256), lambda i: (0, i))
    spec_w = pl.BlockSpec((D, D), lambda i: (0, 0))
    spec_vec = pl.BlockSpec((1, D), lambda i: (0, 0))
    return pl.pallas_call(
        _qkv_kernel,
        grid=(n_blk,),
        in_specs=[spec_rows, spec_w, spec_w, spec_w, spec_vec, spec_vec],
        out_specs=[spec_cols, spec_cols, spec_cols],
        out_shape=[jax.ShapeDtypeStruct((D, NTOK), _F32)] * 3,
        scratch_shapes=[pltpu.VMEM((D, D), jnp.bfloat16)] * 3,
    )(x2d, Wq, Wk, Wv, g.reshape(1, D), b.reshape(1, D))


def _attn_kernel(q_ref, k_ref, v_ref, o_ref):
    s = lax.dot_general((q_ref[...] * (1.0 / 8.0)).astype(jnp.bfloat16),
                        k_ref[...].astype(jnp.bfloat16),
                        (((0,), (0,)), ((), ())),
                        preferred_element_type=_F32).astype(jnp.bfloat16)
    m = jnp.max(s, axis=1, keepdims=True)
    p = jnp.exp(s - m)
    l = jnp.sum(p, axis=1, keepdims=True, dtype=_F32)
    o = lax.dot_general(v_ref[...].astype(jnp.bfloat16), p,
                        (((1,), (1,)), ((), ())),
                        preferred_element_type=_F32)
    o_ref[...] = o * jnp.transpose(1.0 / l)


def _attention(q, k, v):
    QB = 2048
    nqb = 2048 // QB
    return pl.pallas_call(
        _attn_kernel,
        grid=(2, H, nqb),
        in_specs=[
            pl.BlockSpec((DH, QB), lambda bb, hh, qq: (hh, bb * nqb + qq)),
            pl.BlockSpec((DH, 2048), lambda bb, hh, qq: (hh, bb)),
            pl.BlockSpec((DH, 2048), lambda bb, hh, qq: (hh, bb)),
        ],
        out_specs=pl.BlockSpec((DH, QB), lambda bb, hh, qq: (hh, bb * nqb + qq)),
        out_shape=jax.ShapeDtypeStruct((D, NTOK), _F32),
    )(q, k, v)


def _post_kernel(a_ref, x_ref, wo_ref, rw_ref, g_ref, b_ref,
                 x2_ref, mi_ref, lg_ref, wob):
    @pl.when(pl.program_id(0) == 0)
    def _cast():
        wob[...] = wo_ref[...].astype(jnp.bfloat16)

    i = pl.program_id(0)
    ablk = a_ref[:, pl.ds(i * 256, 256)]
    x2 = x_ref[...] + lax.dot_general(ablk.astype(jnp.bfloat16),
                                      wob[...],
                                      (((0,), (0,)), ((), ())),
                                      preferred_element_type=_F32)
    x2_ref[...] = x2
    mu = jnp.mean(x2, axis=1, keepdims=True)
    d = x2 - mu
    var = jnp.mean(d * d, axis=1, keepdims=True)
    mi = d * lax.rsqrt(var + 1e-5) * g_ref[...] + b_ref[...]
    mi_ref[...] = mi
    lg_ref[...] = jnp.dot(mi, rw_ref[...], preferred_element_type=_F32)


def _post_attn(attn, x2d, Wo, router_w, g, b):
    n_blk = NTOK // 256
    spec_rows = pl.BlockSpec((256, D), lambda i: (i, 0))
    return pl.pallas_call(
        _post_kernel,
        grid=(n_blk,),
        in_specs=[
            pl.BlockSpec((D, NTOK), lambda i: (0, 0)),
            spec_rows,
            pl.BlockSpec((D, D), lambda i: (0, 0)),
            pl.BlockSpec((D, E), lambda i: (0, 0)),
            pl.BlockSpec((1, D), lambda i: (0, 0)),
            pl.BlockSpec((1, D), lambda i: (0, 0)),
        ],
        out_specs=[spec_rows, spec_rows, pl.BlockSpec((256, E), lambda i: (i, 0))],
        out_shape=[
            jax.ShapeDtypeStruct((NTOK, D), _F32),
            jax.ShapeDtypeStruct((NTOK, D), _F32),
            jax.ShapeDtypeStruct((NTOK, E), _F32),
        ],
        scratch_shapes=[pltpu.VMEM((D, D), jnp.bfloat16)],
    )(attn, x2d, Wo, router_w, g.reshape(1, D), b.reshape(1, D))


def _route_kernel(lg_ref, pos_ref, e_ref, rs_ref, lo_ref, hi_ref):
    nblk = NTOK // TB
    lane = lax.broadcasted_iota(jnp.int32, (TB, E), 1).astype(_F32)
    ii = lax.broadcasted_iota(jnp.int32, (TB, TB), 0)
    jj = lax.broadcasted_iota(jnp.int32, (TB, TB), 1)
    tril = (jj < ii).astype(_F32)
    carry = jnp.zeros((1, E), _F32)
    ranks = []
    onehots = []
    for bi in range(nblk):
        lg = lg_ref[pl.ds(bi * TB, TB), :]
        m = jnp.max(lg, axis=1, keepdims=True)
        cand = jnp.where(lg >= m, lane, 1e9)
        eid = jnp.min(cand, axis=1, keepdims=True)
        oh = (lane == eid).astype(_F32)
        excl = jnp.dot(tril, oh, preferred_element_type=_F32) + carry
        ranks.append(jnp.sum(excl * oh, axis=1, keepdims=True))
        onehots.append(oh)
        carry = carry + jnp.sum(oh, axis=0, keepdims=True)
    counts = carry
    ei = lax.broadcasted_iota(jnp.int32, (E, E), 0)
    ej = lax.broadcasted_iota(jnp.int32, (E, E), 1)
    upper = (ei < ej).astype(_F32)
    offs = jnp.dot(counts, upper, preferred_element_type=_F32)
    for bi in range(nblk):
        off_tok = jnp.sum(onehots[bi] * offs, axis=1, keepdims=True)
        pos_ref[pl.ds(bi * TB, TB), :] = (ranks[bi] + off_tok).astype(jnp.int32)
    astart = jnp.floor(offs * 0.125) * 8.0
    span = (offs - astart) + counts
    nb = jnp.floor((span + (TB - 1)) * (1.0 / TB))
    boff = jnp.dot(nb, upper, preferred_element_type=_F32)
    g_col = lax.broadcasted_iota(jnp.int32, (NG, 1), 0).astype(_F32)
    ind = jnp.logical_and(g_col >= boff, g_col < boff + nb).astype(_F32)
    erow = lax.broadcasted_iota(jnp.int32, (1, E), 1).astype(_F32)
    cnt_g = jnp.sum(ind * counts, axis=1, keepdims=True)
    offs_g = jnp.sum(ind * offs, axis=1, keepdims=True)
    astart_g = jnp.sum(ind * astart, axis=1, keepdims=True)
    boff_g = jnp.sum(ind * boff, axis=1, keepdims=True)
    e_g = jnp.sum(ind * erow, axis=1, keepdims=True)
    k_g = g_col - boff_g
    win = astart_g + TB * k_g
    rstart = jnp.minimum(win, float(NTOK - TB))
    lo = jnp.maximum(offs_g, win) - rstart
    hi = jnp.minimum(offs_g + cnt_g, win + TB) - rstart
    hi = jnp.maximum(hi, lo)
    e_ref[...] = e_g.astype(jnp.int32)
    rs_ref[...] = rstart.astype(jnp.int32)
    lo_ref[...] = lo.astype(jnp.int32)
    hi_ref[...] = hi.astype(jnp.int32)


def _route(logits):
    return pl.pallas_call(
        _route_kernel,
        out_shape=[
            jax.ShapeDtypeStruct((NTOK, 1), jnp.int32),
            jax.ShapeDtypeStruct((NG, 1), jnp.int32),
            jax.ShapeDtypeStruct((NG, 1), jnp.int32),
            jax.ShapeDtypeStruct((NG, 1), jnp.int32),
            jax.ShapeDtypeStruct((NG, 1), jnp.int32),
        ],
    )(logits)


def _sc_scatter_rows(src, pos):
    mesh = plsc.VectorSubcoreMesh(core_axis_name="c", subcore_axis_name="s")

    def body(src_hbm, pos_hbm, out_hbm, idx_v, rows_v, sem):
        wid = lax.axis_index("s") * SC_NC + lax.axis_index("c")
        base = wid * SC_ROWS
        pltpu.sync_copy(pos_hbm.at[pl.ds(base, SC_ROWS)], idx_v)
        pltpu.sync_copy(src_hbm.at[pl.ds(base, SC_ROWS)], rows_v)
        pltpu.async_copy(rows_v, out_hbm.at[idx_v], sem).wait()

    return pl.kernel(
        body,
        out_type=jax.ShapeDtypeStruct((NTOK, D), _F32),
        mesh=mesh,
        scratch_types=[
            pltpu.VMEM((SC_ROWS,), jnp.int32),
            pltpu.VMEM((SC_ROWS, D), _F32),
            pltpu.SemaphoreType.DMA,
        ],
    )(src, pos)


def _sc_gather_rows(table, pos):
    mesh = plsc.VectorSubcoreMesh(core_axis_name="c", subcore_axis_name="s")

    def body(tab_hbm, pos_hbm, out_hbm, idx_v, rows_v, sem):
        wid = lax.axis_index("s") * SC_NC + lax.axis_index("c")
        base = wid * SC_ROWS
        pltpu.sync_copy(pos_hbm.at[pl.ds(base, SC_ROWS)], idx_v)
        pltpu.async_copy(tab_hbm.at[idx_v], rows_v, sem).wait()
        pltpu.sync_copy(rows_v, out_hbm.at[pl.ds(base, SC_ROWS)])

    return pl.kernel(
        body,
        out_type=jax.ShapeDtypeStruct((NTOK, D), _F32),
        mesh=mesh,
        scratch_types=[
            pltpu.VMEM((SC_ROWS,), jnp.int32),
            pltpu.VMEM((SC_ROWS, D), _F32),
            pltpu.SemaphoreType.DMA,
        ],
    )(table, pos)


def _ffn_kernel(e_ref, rs_ref, lo_ref, hi_ref,
                ts_ref, w1_ref, b1_ref, w2_ref, b2_ref, out_ref):
    g = pl.program_id(0)

    @pl.when(g == 0)
    def _init():
        out_ref[...] = jnp.zeros((NTOK, D), _F32)

    @pl.when(hi_ref[g] > lo_ref[g])
    def _body():
        rs = pl.multiple_of(rs_ref[g], 8)
        xblk = ts_ref[pl.ds(rs, TB), :].astype(jnp.bfloat16)
        h = jnp.dot(xblk, w1_ref[0].astype(jnp.bfloat16),
                    preferred_element_type=_F32) + b1_ref[0]
        h = jax.nn.gelu(h).astype(jnp.bfloat16)
        y = jnp.dot(h, w2_ref[0].astype(jnp.bfloat16),
                    preferred_element_type=_F32) + b2_ref[0]
        rows = lax.broadcasted_iota(jnp.int32, (TB, 1), 0)
        mask = jnp.logical_and(rows >= lo_ref[g], rows < hi_ref[g])
        y = jnp.where(mask, y, 0.0)
        out_ref[pl.ds(rs, TB), :] = out_ref[pl.ds(rs, TB), :] + y


def _grouped_ffn(ts, We1, be1, We2, be2, e_arr, rs_arr, lo_arr, hi_arr):
    grid_spec = pltpu.PrefetchScalarGridSpec(
        num_scalar_prefetch=4,
        grid=(NG,),
        in_specs=[
            pl.BlockSpec((NTOK, D), lambda g, e, r, lo, hi: (0, 0)),
            pl.BlockSpec((1, D, F), lambda g, e, r, lo, hi: (e[g], 0, 0)),
            pl.BlockSpec((1, 1, F), lambda g, e, r, lo, hi: (e[g], 0, 0)),
            pl.BlockSpec((1, F, D), lambda g, e, r, lo, hi: (e[g], 0, 0)),
            pl.BlockSpec((1, 1, D), lambda g, e, r, lo, hi: (e[g], 0, 0)),
        ],
        out_specs=pl.BlockSpec((NTOK, D), lambda g, e, r, lo, hi: (0, 0)),
    )
    return pl.pallas_call(
        _ffn_kernel,
        grid_spec=grid_spec,
        out_shape=jax.ShapeDtypeStruct((NTOK, D), _F32),
        compiler_params=pltpu.CompilerParams(vmem_limit_bytes=100 * 2**20),
    )(e_arr, rs_arr, lo_arr, hi_arr, ts,
      We1, be1.reshape(E, 1, F), We2, be2.reshape(E, 1, D))


def _final_kernel(x2_ref, gth_ref, lg_ref, out_ref):
    lg = lg_ref[...]
    m = jnp.max(lg, axis=1, keepdims=True)
    w = 1.0 / jnp.sum(jnp.exp(lg - m), axis=1, keepdims=True)
    out_ref[...] = x2_ref[...] + w * gth_ref[...]


def _finalize(x2, gathered, logits):
    n_blk = NTOK // 256
    spec_rows = pl.BlockSpec((256, D), lambda i: (i, 0))
    return pl.pallas_call(
        _final_kernel,
        grid=(n_blk,),
        in_specs=[spec_rows, spec_rows, pl.BlockSpec((256, E), lambda i: (i, 0))],
        out_specs=spec_rows,
        out_shape=jax.ShapeDtypeStruct((NTOK, D), _F32),
    )(x2, gathered, logits)


def kernel(x, Wq, Wk, Wv, Wo, ln1_g, ln1_b, ln2_g, ln2_b,
           router_w, We1, be1, We2, be2):
    x2d = x.reshape(NTOK, D)
    q, k, v = _qkv(x2d, Wq, Wk, Wv, ln1_g, ln1_b)
    attn = _attention(q, k, v)
    x2, moe_in, logits = _post_attn(attn, x2d, Wo, router_w, ln2_g, ln2_b)
    pos, e_t, rs_t, lo_t, hi_t = _route(logits)
    pos1d = pos.reshape(NTOK)
    ts = _sc_scatter_rows(moe_in, pos1d)
    ffn = _grouped_ffn(ts, We1, be1, We2, be2,
                       e_t.reshape(NG), rs_t.reshape(NG),
                       lo_t.reshape(NG), hi_t.reshape(NG))
    gathered = _sc_gather_rows(ffn, pos1d)
    out = _finalize(x2, gathered, logits)
    return out.reshape(2, 2048, D)


# speedup vs baseline: 1.0097x; 1.0062x over previous
import jax
import jax.numpy as jnp
from jax import lax
from jax.experimental import pallas as pl
from jax.experimental.pallas import tpu as pltpu
from jax.experimental.pallas import tpu_sc as plsc

D = 768
H = 12
DH = 64
E = 64
F = 3072
TB = 128
NG = 104
NTOK = 4096

SC_NC = 2
SC_NS = 16
SC_NW = SC_NC * SC_NS
SC_ROWS = NTOK // SC_NW

_F32 = jnp.float32


def _qkv_kernel(x_ref, wq_ref, wk_ref, wv_ref, g_ref, b_ref,
                q_ref, k_ref, v_ref, wqb, wkb, wvb):
    @pl.when(pl.program_id(0) == 0)
    def _cast():
        wqb[...] = wq_ref[...].astype(jnp.bfloat16)
        wkb[...] = wk_ref[...].astype(jnp.bfloat16)
        wvb[...] = wv_ref[...].astype(jnp.bfloat16)

    x = x_ref[...]
    mu = jnp.mean(x, axis=1, keepdims=True)
    d = x - mu
    var = jnp.mean(d * d, axis=1, keepdims=True)
    h = d * lax.rsqrt(var + 1e-5) * g_ref[...] + b_ref[...]
    hb = h.astype(jnp.bfloat16)
    q_ref[...] = lax.dot_general(wqb[...], hb, (((0,), (1,)), ((), ())),
                                 preferred_element_type=_F32).astype(jnp.bfloat16)
    k_ref[...] = lax.dot_general(wkb[...], hb, (((0,), (1,)), ((), ())),
                                 preferred_element_type=_F32).astype(jnp.bfloat16)
    v_ref[...] = lax.dot_general(wvb[...], hb, (((0,), (1,)), ((), ())),
                                 preferred_element_type=_F32).astype(jnp.bfloat16)


def _qkv(x2d, Wq, Wk, Wv, g, b):
    n_blk = NTOK // 256
    spec_rows = pl.BlockSpec((256, D), lambda i: (i, 0))
    spec_cols = pl.BlockSpec((D, 256), lambda i: (0, i))
    spec_w = pl.BlockSpec((D, D), lambda i: (0, 0))
    spec_vec = pl.BlockSpec((1, D), lambda i: (0, 0))
    return pl.pallas_call(
        _qkv_kernel,
        grid=(n_blk,),
        in_specs=[spec_rows, spec_w, spec_w, spec_w, spec_vec, spec_vec],
        out_specs=[spec_cols, spec_cols, spec_cols],
        out_shape=[jax.ShapeDtypeStruct((D, NTOK), jnp.bfloat16)] * 3,
        scratch_shapes=[pltpu.VMEM((D, D), jnp.bfloat16)] * 3,
    )(x2d, Wq, Wk, Wv, g.reshape(1, D), b.reshape(1, D))


def _attn_kernel(q_ref, k_ref, v_ref, o_ref):
    s = lax.dot_general(q_ref[...] * jnp.bfloat16(1.0 / 8.0), k_ref[...],
                        (((0,), (0,)), ((), ())),
                        preferred_element_type=_F32).astype(jnp.bfloat16)
    m = jnp.max(s, axis=1, keepdims=True)
    p = jnp.exp(s - m)
    l = jnp.sum(p, axis=1, keepdims=True, dtype=_F32)
    o = lax.dot_general(v_ref[...], p, (((1,), (1,)), ((), ())),
                        preferred_element_type=_F32)
    o_ref[...] = (o * jnp.transpose(1.0 / l)).astype(jnp.bfloat16)


def _attention(q, k, v):
    QB = 2048
    nqb = 2048 // QB
    return pl.pallas_call(
        _attn_kernel,
        grid=(2, H, nqb),
        in_specs=[
            pl.BlockSpec((DH, QB), lambda bb, hh, qq: (hh, bb * nqb + qq)),
            pl.BlockSpec((DH, 2048), lambda bb, hh, qq: (hh, bb)),
            pl.BlockSpec((DH, 2048), lambda bb, hh, qq: (hh, bb)),
        ],
        out_specs=pl.BlockSpec((DH, QB), lambda bb, hh, qq: (hh, bb * nqb + qq)),
        out_shape=jax.ShapeDtypeStruct((D, NTOK), jnp.bfloat16),
    )(q, k, v)


def _post_kernel(a_ref, x_ref, wo_ref, rw_ref, g_ref, b_ref,
                 x2_ref, mi_ref, lg_ref, wob):
    @pl.when(pl.program_id(0) == 0)
    def _cast():
        wob[...] = wo_ref[...].astype(jnp.bfloat16)

    i = pl.program_id(0)
    ablk = a_ref[:, pl.ds(i * 256, 256)]
    x2 = x_ref[...] + lax.dot_general(ablk,
                                      wob[...],
                                      (((0,), (0,)), ((), ())),
                                      preferred_element_type=_F32)
    x2_ref[...] = x2
    mu = jnp.mean(x2, axis=1, keepdims=True)
    d = x2 - mu
    var = jnp.mean(d * d, axis=1, keepdims=True)
    mi = d * lax.rsqrt(var + 1e-5) * g_ref[...] + b_ref[...]
    mi_ref[...] = mi
    lg_ref[...] = jnp.dot(mi, rw_ref[...], preferred_element_type=_F32)


def _post_attn(attn, x2d, Wo, router_w, g, b):
    n_blk = NTOK // 256
    spec_rows = pl.BlockSpec((256, D), lambda i: (i, 0))
    return pl.pallas_call(
        _post_kernel,
        grid=(n_blk,),
        in_specs=[
            pl.BlockSpec((D, NTOK), lambda i: (0, 0)),
            spec_rows,
            pl.BlockSpec((D, D), lambda i: (0, 0)),
            pl.BlockSpec((D, E), lambda i: (0, 0)),
            pl.BlockSpec((1, D), lambda i: (0, 0)),
            pl.BlockSpec((1, D), lambda i: (0, 0)),
        ],
        out_specs=[spec_rows, spec_rows, pl.BlockSpec((256, E), lambda i: (i, 0))],
        out_shape=[
            jax.ShapeDtypeStruct((NTOK, D), _F32),
            jax.ShapeDtypeStruct((NTOK, D), _F32),
            jax.ShapeDtypeStruct((NTOK, E), _F32),
        ],
        scratch_shapes=[pltpu.VMEM((D, D), jnp.bfloat16)],
    )(attn, x2d, Wo, router_w, g.reshape(1, D), b.reshape(1, D))


def _route_kernel(lg_ref, pos_ref, e_ref, rs_ref, lo_ref, hi_ref):
    nblk = NTOK // TB
    lane = lax.broadcasted_iota(jnp.int32, (TB, E), 1).astype(_F32)
    ii = lax.broadcasted_iota(jnp.int32, (TB, TB), 0)
    jj = lax.broadcasted_iota(jnp.int32, (TB, TB), 1)
    tril = (jj < ii).astype(_F32)
    carry = jnp.zeros((1, E), _F32)
    ranks = []
    onehots = []
    for bi in range(nblk):
        lg = lg_ref[pl.ds(bi * TB, TB), :]
        m = jnp.max(lg, axis=1, keepdims=True)
        cand = jnp.where(lg >= m, lane, 1e9)
        eid = jnp.min(cand, axis=1, keepdims=True)
        oh = (lane == eid).astype(_F32)
        excl = jnp.dot(tril, oh, preferred_element_type=_F32) + carry
        ranks.append(jnp.sum(excl * oh, axis=1, keepdims=True))
        onehots.append(oh)
        carry = carry + jnp.sum(oh, axis=0, keepdims=True)
    counts = carry
    ei = lax.broadcasted_iota(jnp.int32, (E, E), 0)
    ej = lax.broadcasted_iota(jnp.int32, (E, E), 1)
    upper = (ei < ej).astype(_F32)
    offs = jnp.dot(counts, upper, preferred_element_type=_F32)
    for bi in range(nblk):
        off_tok = jnp.sum(onehots[bi] * offs, axis=1, keepdims=True)
        pos_ref[pl.ds(bi * TB, TB), :] = (ranks[bi] + off_tok).astype(jnp.int32)
    astart = jnp.floor(offs * 0.125) * 8.0
    span = (offs - astart) + counts
    nb = jnp.floor((span + (TB - 1)) * (1.0 / TB))
    boff = jnp.dot(nb, upper, preferred_element_type=_F32)
    g_col = lax.broadcasted_iota(jnp.int32, (NG, 1), 0).astype(_F32)
    ind = jnp.logical_and(g_col >= boff, g_col < boff + nb).astype(_F32)
    erow = lax.broadcasted_iota(jnp.int32, (1, E), 1).astype(_F32)
    cnt_g = jnp.sum(ind * counts, axis=1, keepdims=True)
    offs_g = jnp.sum(ind * offs, axis=1, keepdims=True)
    astart_g = jnp.sum(ind * astart, axis=1, keepdims=True)
    boff_g = jnp.sum(ind * boff, axis=1, keepdims=True)
    e_g = jnp.sum(ind * erow, axis=1, keepdims=True)
    k_g = g_col - boff_g
    win = astart_g + TB * k_g
    rstart = jnp.minimum(win, float(NTOK - TB))
    lo = jnp.maximum(offs_g, win) - rstart
    hi = jnp.minimum(offs_g + cnt_g, win + TB) - rstart
    hi = jnp.maximum(hi, lo)
    e_ref[...] = e_g.astype(jnp.int32)
    rs_ref[...] = rstart.astype(jnp.int32)
    lo_ref[...] = lo.astype(jnp.int32)
    hi_ref[...] = hi.astype(jnp.int32)


def _route(logits):
    return pl.pallas_call(
        _route_kernel,
        out_shape=[
            jax.ShapeDtypeStruct((NTOK, 1), jnp.int32),
            jax.ShapeDtypeStruct((NG, 1), jnp.int32),
            jax.ShapeDtypeStruct((NG, 1), jnp.int32),
            jax.ShapeDtypeStruct((NG, 1), jnp.int32),
            jax.ShapeDtypeStruct((NG, 1), jnp.int32),
        ],
    )(logits)


def _sc_scatter_rows(src, pos):
    mesh = plsc.VectorSubcoreMesh(core_axis_name="c", subcore_axis_name="s")

    def body(src_hbm, pos_hbm, out_hbm, idx_v, rows_v, sem):
        wid = lax.axis_index("s") * SC_NC + lax.axis_index("c")
        base = wid * SC_ROWS
        pltpu.sync_copy(pos_hbm.at[pl.ds(base, SC_ROWS)], idx_v)
        pltpu.sync_copy(src_hbm.at[pl.ds(base, SC_ROWS)], rows_v)
        pltpu.async_copy(rows_v, out_hbm.at[idx_v], sem).wait()

    return pl.kernel(
        body,
        out_type=jax.ShapeDtypeStruct((NTOK, D), _F32),
        mesh=mesh,
        scratch_types=[
            pltpu.VMEM((SC_ROWS,), jnp.int32),
            pltpu.VMEM((SC_ROWS, D), _F32),
            pltpu.SemaphoreType.DMA,
        ],
    )(src, pos)


def _sc_gather_rows(table, pos):
    mesh = plsc.VectorSubcoreMesh(core_axis_name="c", subcore_axis_name="s")

    def body(tab_hbm, pos_hbm, out_hbm, idx_v, rows_v, sem):
        wid = lax.axis_index("s") * SC_NC + lax.axis_index("c")
        base = wid * SC_ROWS
        pltpu.sync_copy(pos_hbm.at[pl.ds(base, SC_ROWS)], idx_v)
        pltpu.async_copy(tab_hbm.at[idx_v], rows_v, sem).wait()
        pltpu.sync_copy(rows_v, out_hbm.at[pl.ds(base, SC_ROWS)])

    return pl.kernel(
        body,
        out_type=jax.ShapeDtypeStruct((NTOK, D), _F32),
        mesh=mesh,
        scratch_types=[
            pltpu.VMEM((SC_ROWS,), jnp.int32),
            pltpu.VMEM((SC_ROWS, D), _F32),
            pltpu.SemaphoreType.DMA,
        ],
    )(table, pos)


def _ffn_kernel(e_ref, rs_ref, lo_ref, hi_ref,
                ts_ref, w1_ref, b1_ref, w2_ref, b2_ref, out_ref):
    g = pl.program_id(0)

    @pl.when(g == 0)
    def _init():
        out_ref[...] = jnp.zeros((NTOK, D), _F32)

    @pl.when(hi_ref[g] > lo_ref[g])
    def _body():
        rs = pl.multiple_of(rs_ref[g], 8)
        xblk = ts_ref[pl.ds(rs, TB), :].astype(jnp.bfloat16)
        h = jnp.dot(xblk, w1_ref[0].astype(jnp.bfloat16),
                    preferred_element_type=_F32) + b1_ref[0]
        h = jax.nn.gelu(h).astype(jnp.bfloat16)
        y = jnp.dot(h, w2_ref[0].astype(jnp.bfloat16),
                    preferred_element_type=_F32) + b2_ref[0]
        rows = lax.broadcasted_iota(jnp.int32, (TB, 1), 0)
        mask = jnp.logical_and(rows >= lo_ref[g], rows < hi_ref[g])
        y = jnp.where(mask, y, 0.0)
        out_ref[pl.ds(rs, TB), :] = out_ref[pl.ds(rs, TB), :] + y


def _grouped_ffn(ts, We1, be1, We2, be2, e_arr, rs_arr, lo_arr, hi_arr):
    grid_spec = pltpu.PrefetchScalarGridSpec(
        num_scalar_prefetch=4,
        grid=(NG,),
        in_specs=[
            pl.BlockSpec((NTOK, D), lambda g, e, r, lo, hi: (0, 0)),
            pl.BlockSpec((1, D, F), lambda g, e, r, lo, hi: (e[g], 0, 0)),
            pl.BlockSpec((1, 1, F), lambda g, e, r, lo, hi: (e[g], 0, 0)),
            pl.BlockSpec((1, F, D), lambda g, e, r, lo, hi: (e[g], 0, 0)),
            pl.BlockSpec((1, 1, D), lambda g, e, r, lo, hi: (e[g], 0, 0)),
        ],
        out_specs=pl.BlockSpec((NTOK, D), lambda g, e, r, lo, hi: (0, 0)),
    )
    return pl.pallas_call(
        _ffn_kernel,
        grid_spec=grid_spec,
        out_shape=jax.ShapeDtypeStruct((NTOK, D), _F32),
        compiler_params=pltpu.CompilerParams(vmem_limit_bytes=100 * 2**20),
    )(e_arr, rs_arr, lo_arr, hi_arr, ts,
      We1, be1.reshape(E, 1, F), We2, be2.reshape(E, 1, D))


def _final_kernel(x2_ref, gth_ref, lg_ref, out_ref):
    lg = lg_ref[...]
    m = jnp.max(lg, axis=1, keepdims=True)
    w = 1.0 / jnp.sum(jnp.exp(lg - m), axis=1, keepdims=True)
    out_ref[...] = x2_ref[...] + w * gth_ref[...]


def _finalize(x2, gathered, logits):
    n_blk = NTOK // 256
    spec_rows = pl.BlockSpec((256, D), lambda i: (i, 0))
    return pl.pallas_call(
        _final_kernel,
        grid=(n_blk,),
        in_specs=[spec_rows, spec_rows, pl.BlockSpec((256, E), lambda i: (i, 0))],
        out_specs=spec_rows,
        out_shape=jax.ShapeDtypeStruct((NTOK, D), _F32),
    )(x2, gathered, logits)


def kernel(x, Wq, Wk, Wv, Wo, ln1_g, ln1_b, ln2_g, ln2_b,
           router_w, We1, be1, We2, be2):
    x2d = x.reshape(NTOK, D)
    q, k, v = _qkv(x2d, Wq, Wk, Wv, ln1_g, ln1_b)
    attn = _attention(q, k, v)
    x2, moe_in, logits = _post_attn(attn, x2d, Wo, router_w, ln2_g, ln2_b)
    pos, e_t, rs_t, lo_t, hi_t = _route(logits)
    pos1d = pos.reshape(NTOK)
    ts = _sc_scatter_rows(moe_in, pos1d)
    ffn = _grouped_ffn(ts, We1, be1, We2, be2,
                       e_t.reshape(NG), rs_t.reshape(NG),
                       lo_t.reshape(NG), hi_t.reshape(NG))
    gathered = _sc_gather_rows(ffn, pos1d)
    out = _finalize(x2, gathered, logits)
    return out.reshape(2, 2048, D)
